# Initial kernel scaffold; baseline (speedup 1.0000x reference)
import jax
import jax.numpy as jnp
from jax.experimental import pallas as pl


def kernel(x, Wg, W1, W3, W2):
    raise NotImplementedError("write your pallas kernel here")


# speedup vs baseline: 2.5666x; 2.5666x over previous
import functools

import jax
import jax.numpy as jnp
from jax.experimental import pallas as pl

E = 8
K = 2


def _gate_kernel(x_ref, wg_ref, ew_ref):
    scores = jnp.dot(x_ref[...], wg_ref[...].T, preferred_element_type=jnp.float32)
    s, e = scores.shape
    idx = jax.lax.broadcasted_iota(jnp.int32, (s, e), 1)
    v1 = jnp.max(scores, axis=1, keepdims=True)
    i1 = jnp.min(jnp.where(scores == v1, idx, e), axis=1, keepdims=True)
    masked = jnp.where(idx == i1, -jnp.inf, scores)
    v2 = jnp.max(masked, axis=1, keepdims=True)
    i2 = jnp.min(jnp.where(masked == v2, idx, e), axis=1, keepdims=True)
    w1 = 1.0 / (1.0 + jnp.exp(v2 - v1))
    w2 = 1.0 - w1
    ew_ref[...] = jnp.where(idx == i1, w1, 0.0) + jnp.where(idx == i2, w2, 0.0)


def _ffn_kernel(x_ref, w1_ref, w3_ref, w2_ref, ew_ref, y_ref):
    e = pl.program_id(0)
    h = pl.program_id(1)

    @pl.when((e == 0) & (h == 0))
    def _():
        y_ref[...] = jnp.zeros_like(y_ref)

    xb = x_ref[...]
    g = jnp.dot(xb, w1_ref[0].T, preferred_element_type=jnp.float32)
    u = jnp.dot(xb, w3_ref[0].T, preferred_element_type=jnp.float32)
    a = (g * jax.nn.sigmoid(g) * u).astype(jnp.bfloat16)
    o = jnp.dot(a, w2_ref[0].T, preferred_element_type=jnp.float32)
    ew = ew_ref[...]
    lane = jax.lax.broadcasted_iota(jnp.int32, ew.shape, 1)
    ewcol = jnp.sum(jnp.where(lane == e, ew, 0.0), axis=1, keepdims=True)
    y_ref[...] += ewcol * o


@functools.partial(jax.jit, static_argnames=())
def kernel(x, Wg, W1, W3, W2):
    orig_shape = x.shape
    dim = orig_shape[-1]
    xf = x.reshape(-1, dim)
    s = xf.shape[0]
    num_e, hid, _ = W1.shape

    xb = xf.astype(jnp.bfloat16)
    w1b = W1.astype(jnp.bfloat16)
    w3b = W3.astype(jnp.bfloat16)
    w2b = W2.astype(jnp.bfloat16)

    ew = pl.pallas_call(
        _gate_kernel,
        out_shape=jax.ShapeDtypeStruct((s, num_e), jnp.float32),
    )(xf, Wg)

    hb = 256
    nh = hid // hb

    y = pl.pallas_call(
        _ffn_kernel,
        grid=(num_e, nh),
        in_specs=[
            pl.BlockSpec((s, dim), lambda e, h: (0, 0)),
            pl.BlockSpec((1, hb, dim), lambda e, h: (e, h, 0)),
            pl.BlockSpec((1, hb, dim), lambda e, h: (e, h, 0)),
            pl.BlockSpec((1, dim, hb), lambda e, h: (e, 0, h)),
            pl.BlockSpec((s, num_e), lambda e, h: (0, 0)),
        ],
        out_specs=pl.BlockSpec((s, dim), lambda e, h: (0, 0)),
        out_shape=jax.ShapeDtypeStruct((s, dim), jnp.float32),
    )(xb, w1b, w3b, w2b, ew)

    return y.reshape(orig_shape)
